# baseline (speedup 1.0000x reference)
import functools

import jax
import jax.numpy as jnp
from jax.experimental import pallas as pl
from jax.experimental.pallas import tpu as pltpu


def _gcn_kernel(t_hbm, x_ref, f_ref, o_hbm, xf_ref, buf_ref, ob_ref,
                in_sem, out_sem, *, bm: int, nbuf: int, nchunk: int):
    def in_copy(chunk, slot):
        return pltpu.make_async_copy(
            t_hbm.at[pl.ds(chunk * bm, bm), :], buf_ref.at[slot],
            in_sem.at[slot])

    def out_copy(chunk, slot):
        return pltpu.make_async_copy(
            ob_ref.at[slot], o_hbm.at[pl.ds(chunk * bm, bm), :],
            out_sem.at[slot])

    for i in range(nbuf):
        in_copy(i, i).start()

    xf_ref[...] = jnp.dot(
        x_ref[...], f_ref[...], preferred_element_type=jnp.float32)

    def step(j, carry):
        slot = jax.lax.rem(j, nbuf)
        oslot = jax.lax.rem(j, 2)
        in_copy(j, slot).wait()

        @pl.when(j >= 2)
        def _wait_prev_out():
            out_copy(j - 2, oslot).wait()

        ob_ref[oslot] = jnp.dot(
            buf_ref[slot], xf_ref[...], preferred_element_type=jnp.float32)
        out_copy(j, oslot).start()

        @pl.when(j + nbuf < nchunk)
        def _next_in():
            in_copy(j + nbuf, slot).start()

        return carry

    jax.lax.fori_loop(0, nchunk, step, 0)
    out_copy(nchunk - 2, (nchunk - 2) % 2).wait()
    out_copy(nchunk - 1, (nchunk - 1) % 2).wait()


def kernel(x, transform, filters):
    n, n_feat = x.shape
    n_filt = filters.shape[1]

    bm = 200
    nbuf = 4
    nchunk = n // bm

    return pl.pallas_call(
        functools.partial(_gcn_kernel, bm=bm, nbuf=nbuf, nchunk=nchunk),
        in_specs=[
            pl.BlockSpec(memory_space=pl.ANY),
            pl.BlockSpec(memory_space=pltpu.MemorySpace.VMEM),
            pl.BlockSpec(memory_space=pltpu.MemorySpace.VMEM),
        ],
        out_specs=pl.BlockSpec(memory_space=pl.ANY),
        out_shape=jax.ShapeDtypeStruct((n, n_filt), jnp.float32),
        scratch_shapes=[
            pltpu.VMEM((n, n_filt), jnp.float32),
            pltpu.VMEM((nbuf, bm, n), jnp.float32),
            pltpu.VMEM((2, bm, n_filt), jnp.float32),
            pltpu.SemaphoreType.DMA((nbuf,)),
            pltpu.SemaphoreType.DMA((2,)),
        ],
    )(transform, x, filters)
